# baseline (speedup 1.0000x reference)
import functools

import jax
import jax.numpy as jnp
import numpy as np
from jax import lax
from jax.experimental import pallas as pl
from jax.experimental.pallas import tpu as pltpu
from jax.experimental.pallas import tpu_sc as plsc

VOCAB = 1000000
D = 64
BATCH = 4096
SEQ = 200

NC = 2
NS = 16
NW = NC * NS

BPW = BATCH // NW
NBUF = 4


def _pe_table(max_len, d_emb):
    pos = np.arange(max_len, dtype=np.float64)[:, None]
    i = np.arange(d_emb, dtype=np.float64)[None, :]
    pe = pos / (10000.0 ** (2.0 * i / d_emb))
    pe[0, :] = 0.0
    pe[:, 0::2] = np.sin(pe[:, 0::2])
    pe[:, 1::2] = np.cos(pe[:, 1::2])
    return pe.astype(np.float32)


_TCHUNK = 4096


def _tc_transpose_pad(tt):
    def body(in_ref, out_ref):
        t = in_ref[...]
        out_ref[...] = jnp.concatenate(
            [t.T, jnp.zeros((_TCHUNK, D), jnp.float32)], axis=1)

    return pl.pallas_call(
        body,
        grid=(pl.cdiv(VOCAB, _TCHUNK),),
        in_specs=[pl.BlockSpec((D, _TCHUNK), lambda j: (0, j))],
        out_specs=pl.BlockSpec((_TCHUNK, 2 * D), lambda j: (j, 0)),
        out_shape=jax.ShapeDtypeStruct((VOCAB, 2 * D), jnp.float32),
    )(tt)


def _sc_embed(tpad2, idxt, pe):
    mesh = plsc.VectorSubcoreMesh(core_axis_name="c", subcore_axis_name="s")

    @functools.partial(
        pl.kernel,
        out_type=jax.ShapeDtypeStruct((SEQ, 8, NW, 8 * BPW), jnp.float32),
        mesh=mesh,
        compiler_params=pltpu.CompilerParams(use_tc_tiling_on_sc=False, needs_layout_passes=False),
        scratch_types=[
            pltpu.VMEM((SEQ, BPW), jnp.int32),
            pltpu.VMEM((SEQ * D,), jnp.float32),
            [pltpu.VMEM((BPW, D), jnp.float32) for _ in range(NBUF)],
            pltpu.VMEM((BPW * (D + 1),), jnp.float32),
            [pltpu.VMEM((8, 8 * BPW), jnp.float32) for _ in range(NBUF)],
            [pltpu.SemaphoreType.DMA for _ in range(NBUF)],
            [pltpu.SemaphoreType.DMA for _ in range(NBUF)],
        ],
    )
    def k(tab_hbm, idx_hbm, pe_hbm, out_hbm, idx_v, pe_v, gb, gp, tb,
          gsem, ssem):
        wid = lax.axis_index("s") * NC + lax.axis_index("c")

        pltpu.sync_copy(idx_hbm.at[:, pl.ds(wid * BPW, BPW)], idx_v)
        pltpu.sync_copy(pe_hbm, pe_v)

        @plsc.parallel_loop(0, SEQ)
        def _(l):
            for g in range(BPW // 16):
                s = pl.ds(g * 16, 16)
                idx_v[l, s] = idx_v[l, s] * 2

        def start_gather(l, b):
            pltpu.async_copy(tab_hbm.at[idx_v.at[l]], gb[b], gsem[b])

        def wait_gather(l, b):
            pltpu.make_async_copy(
                tab_hbm.at[idx_v.at[l]], gb[b], gsem[b]).wait()

        def start_scatter(l, b):
            pltpu.async_copy(tb[b], out_hbm.at[l, :, wid], ssem[b])

        def wait_scatter(l, b):
            pltpu.make_async_copy(
                tb[b], out_hbm.at[l, :, wid], ssem[b]).wait()

        iota16 = lax.iota(jnp.int32, 16)
        zeros16 = iota16 * 0
        row_ids = [iota16 + g * 16 for g in range(BPW // 16)]

        row65 = [(iota16 + g * 16) * (D + 1) for g in range(BPW // 16)]

        def transpose_add(l, b):
            src, dst = gb[b], tb[b]
            lD = l * D
            pes4 = [pe_v[pl.ds(lD + q * 16, 16)] for q in range(4)]

            @plsc.parallel_loop(0, BPW, unroll=2)
            def _(r):
                for q in range(4):
                    gp[pl.ds(r * (D + 1) + q * 16, 16)] = (
                        src[r, pl.ds(q * 16, 16)] + pes4[q])

            @plsc.parallel_loop(0, D, unroll=2)
            def _(d):
                d8 = d >> 3
                off = (d & 7) * BPW
                cols = [plsc.load_gather(gp, [row65[g] + d])
                        for g in range(BPW // 16)]
                for g in range(BPW // 16):
                    dst[d8, pl.ds(off + g * 16, 16)] = cols[g]

        for b in range(NBUF):
            start_gather(b, b)

        for b in range(NBUF):
            wait_gather(b, b)
            transpose_add(b, b)
            start_gather(b + NBUF, b)
            start_scatter(b, b)

        @pl.loop(NBUF, SEQ - NBUF, step=NBUF)
        def _(ll):
            for b in range(NBUF):
                l = ll + b
                wait_gather(l, b)
                wait_scatter(l - NBUF, b)
                transpose_add(l, b)
                start_gather(l + NBUF, b)
                start_scatter(l, b)

        for b in range(NBUF):
            l = SEQ - NBUF + b
            wait_gather(l, b)
            wait_scatter(l - NBUF, b)
            transpose_add(l, b)
            start_scatter(l, b)

        for b in range(NBUF):
            wait_scatter(SEQ - NBUF + b, b)

    return k(tpad2, idxt, pe)


_PE = _pe_table(SEQ, D)


def kernel(inputs, table):
    tpad2 = _tc_transpose_pad(table.T).reshape(2 * VOCAB, D)
    idxt = inputs.astype(jnp.int32).T
    pe = jnp.asarray(_PE).reshape(SEQ * D)
    out = _sc_embed(tpad2, idxt, pe)
    out = out.reshape(SEQ, 8, NW, 8, BPW)
    return out.transpose(2, 4, 0, 1, 3).reshape(BATCH, SEQ, D)


# speedup vs baseline: 1.1656x; 1.1656x over previous
import functools

import jax
import jax.numpy as jnp
import numpy as np
from jax import lax
from jax.experimental import pallas as pl
from jax.experimental.pallas import tpu as pltpu
from jax.experimental.pallas import tpu_sc as plsc

VOCAB = 1000000
D = 64
BATCH = 4096
SEQ = 200

NC = 2
NS = 16
NW = NC * NS

BPW = BATCH // NW
NBUF = 4


def _pe_table(max_len, d_emb):
    pos = np.arange(max_len, dtype=np.float64)[:, None]
    i = np.arange(d_emb, dtype=np.float64)[None, :]
    pe = pos / (10000.0 ** (2.0 * i / d_emb))
    pe[0, :] = 0.0
    pe[:, 0::2] = np.sin(pe[:, 0::2])
    pe[:, 1::2] = np.cos(pe[:, 1::2])
    return pe.astype(np.float32)


_TCHUNK = 8192


def _tc_transpose_pad(tt):
    def body(in_ref, out_ref):
        t = in_ref[...]
        out_ref[...] = jnp.concatenate(
            [t.T, jnp.zeros((_TCHUNK, D), jnp.float32)], axis=1)

    return pl.pallas_call(
        body,
        grid=(pl.cdiv(VOCAB, _TCHUNK),),
        in_specs=[pl.BlockSpec((D, _TCHUNK), lambda j: (0, j))],
        out_specs=pl.BlockSpec((_TCHUNK, 2 * D), lambda j: (j, 0)),
        out_shape=jax.ShapeDtypeStruct((VOCAB, 2 * D), jnp.float32),
    )(tt)


def _sc_embed(tpad2, idxt, pe):
    mesh = plsc.VectorSubcoreMesh(core_axis_name="c", subcore_axis_name="s")

    @functools.partial(
        pl.kernel,
        out_type=jax.ShapeDtypeStruct((SEQ, 8, NW, 8 * BPW), jnp.float32),
        mesh=mesh,
        compiler_params=pltpu.CompilerParams(use_tc_tiling_on_sc=False, needs_layout_passes=False),
        scratch_types=[
            pltpu.VMEM((SEQ, BPW), jnp.int32),
            pltpu.VMEM((SEQ * D,), jnp.float32),
            [pltpu.VMEM((BPW, D), jnp.float32) for _ in range(NBUF)],
            pltpu.VMEM((BPW * (D + 1),), jnp.float32),
            [pltpu.VMEM((8, 8 * BPW), jnp.float32) for _ in range(NBUF)],
            [pltpu.SemaphoreType.DMA for _ in range(NBUF)],
            [pltpu.SemaphoreType.DMA for _ in range(NBUF)],
        ],
    )
    def k(tab_hbm, idx_hbm, pe_hbm, out_hbm, idx_v, pe_v, gb, gp, tb,
          gsem, ssem):
        wid = lax.axis_index("s") * NC + lax.axis_index("c")

        pltpu.sync_copy(idx_hbm.at[:, pl.ds(wid * BPW, BPW)], idx_v)
        pltpu.sync_copy(pe_hbm, pe_v)

        @plsc.parallel_loop(0, SEQ)
        def _(l):
            for g in range(BPW // 16):
                s = pl.ds(g * 16, 16)
                idx_v[l, s] = idx_v[l, s] * 2

        def start_gather(l, b):
            pltpu.async_copy(tab_hbm.at[idx_v.at[l]], gb[b], gsem[b])

        def wait_gather(l, b):
            pltpu.make_async_copy(
                tab_hbm.at[idx_v.at[l]], gb[b], gsem[b]).wait()

        def start_scatter(l, b):
            pltpu.async_copy(tb[b], out_hbm.at[l, :, wid], ssem[b])

        def wait_scatter(l, b):
            pltpu.make_async_copy(
                tb[b], out_hbm.at[l, :, wid], ssem[b]).wait()

        iota16 = lax.iota(jnp.int32, 16)
        zeros16 = iota16 * 0
        row_ids = [iota16 + g * 16 for g in range(BPW // 16)]

        row65 = [(iota16 + g * 16) * (D + 1) for g in range(BPW // 16)]

        def transpose_add(l, b):
            src, dst = gb[b], tb[b]
            lD = l * D
            pes4 = [pe_v[pl.ds(lD + q * 16, 16)] for q in range(4)]

            @plsc.parallel_loop(0, BPW, unroll=2)
            def _(r):
                for q in range(4):
                    gp[pl.ds(r * (D + 1) + q * 16, 16)] = (
                        src[r, pl.ds(q * 16, 16)] + pes4[q])

            @plsc.parallel_loop(0, D, unroll=2)
            def _(d):
                d8 = d >> 3
                off = (d & 7) * BPW
                cols = [plsc.load_gather(gp, [row65[g] + d])
                        for g in range(BPW // 16)]
                for g in range(BPW // 16):
                    dst[d8, pl.ds(off + g * 16, 16)] = cols[g]

        for b in range(NBUF):
            start_gather(b, b)

        for b in range(NBUF):
            wait_gather(b, b)
            transpose_add(b, b)
            start_gather(b + NBUF, b)
            start_scatter(b, b)

        @pl.loop(NBUF, SEQ - NBUF, step=NBUF)
        def _(ll):
            for b in range(NBUF):
                l = ll + b
                wait_gather(l, b)
                wait_scatter(l - NBUF, b)
                transpose_add(l, b)
                start_gather(l + NBUF, b)
                start_scatter(l, b)

        for b in range(NBUF):
            l = SEQ - NBUF + b
            wait_gather(l, b)
            wait_scatter(l - NBUF, b)
            transpose_add(l, b)
            start_scatter(l, b)

        for b in range(NBUF):
            wait_scatter(SEQ - NBUF + b, b)

    return k(tpad2, idxt, pe)


_PE = _pe_table(SEQ, D)


def kernel(inputs, table):
    tpad2 = _tc_transpose_pad(table.T).reshape(2 * VOCAB, D)
    idxt = inputs.astype(jnp.int32).T
    pe = jnp.asarray(_PE).reshape(SEQ * D)
    out = _sc_embed(tpad2, idxt, pe)
    out = out.reshape(SEQ, 8, NW, 8, BPW)
    return out.transpose(2, 4, 0, 1, 3).reshape(BATCH, SEQ, D)


# speedup vs baseline: 1.1666x; 1.0008x over previous
import functools

import jax
import jax.numpy as jnp
import numpy as np
from jax import lax
from jax.experimental import pallas as pl
from jax.experimental.pallas import tpu as pltpu
from jax.experimental.pallas import tpu_sc as plsc

VOCAB = 1000000
D = 64
BATCH = 4096
SEQ = 200

NC = 2
NS = 16
NW = NC * NS

BPW = BATCH // NW
NBUF = 4


def _pe_table(max_len, d_emb):
    pos = np.arange(max_len, dtype=np.float64)[:, None]
    i = np.arange(d_emb, dtype=np.float64)[None, :]
    pe = pos / (10000.0 ** (2.0 * i / d_emb))
    pe[0, :] = 0.0
    pe[:, 0::2] = np.sin(pe[:, 0::2])
    pe[:, 1::2] = np.cos(pe[:, 1::2])
    return pe.astype(np.float32)


_TCHUNK = 8192


def _tc_transpose_pad(tt):
    def body(in_ref, out_ref):
        t = in_ref[...]
        out_ref[...] = jnp.concatenate(
            [t.T, jnp.zeros((_TCHUNK, D), jnp.float32)], axis=1)

    return pl.pallas_call(
        body,
        grid=(pl.cdiv(VOCAB, _TCHUNK),),
        in_specs=[pl.BlockSpec((D, _TCHUNK), lambda j: (0, j))],
        out_specs=pl.BlockSpec((_TCHUNK, 2 * D), lambda j: (j, 0)),
        out_shape=jax.ShapeDtypeStruct((VOCAB, 2 * D), jnp.float32),
    )(tt)


def _sc_embed(tpad2, idxt, pe):
    mesh = plsc.VectorSubcoreMesh(core_axis_name="c", subcore_axis_name="s")

    @functools.partial(
        pl.kernel,
        out_type=jax.ShapeDtypeStruct((SEQ, 8, NW, 8 * BPW), jnp.float32),
        mesh=mesh,
        compiler_params=pltpu.CompilerParams(use_tc_tiling_on_sc=False, needs_layout_passes=False),
        scratch_types=[
            pltpu.VMEM((SEQ, BPW), jnp.int32),
            pltpu.VMEM((SEQ * D,), jnp.float32),
            [pltpu.VMEM((BPW, D), jnp.float32) for _ in range(NBUF)],
            pltpu.VMEM((BPW * (D + 1),), jnp.float32),
            [pltpu.VMEM((8, 8 * BPW), jnp.float32) for _ in range(NBUF)],
            [pltpu.SemaphoreType.DMA for _ in range(NBUF)],
            [pltpu.SemaphoreType.DMA for _ in range(NBUF)],
        ],
    )
    def k(tab_hbm, idx_hbm, pe_hbm, out_hbm, idx_v, pe_v, gb, gp, tb,
          gsem, ssem):
        wid = lax.axis_index("s") * NC + lax.axis_index("c")

        pltpu.sync_copy(idx_hbm.at[:, pl.ds(wid * BPW, BPW)], idx_v)
        pltpu.sync_copy(pe_hbm, pe_v)

        def start_gather(l, b):
            pltpu.async_copy(tab_hbm.at[idx_v.at[l]], gb[b], gsem[b])

        def wait_gather(l, b):
            pltpu.make_async_copy(
                tab_hbm.at[idx_v.at[l]], gb[b], gsem[b]).wait()

        def start_scatter(l, b):
            pltpu.async_copy(tb[b], out_hbm.at[l, :, wid], ssem[b])

        def wait_scatter(l, b):
            pltpu.make_async_copy(
                tb[b], out_hbm.at[l, :, wid], ssem[b]).wait()

        iota16 = lax.iota(jnp.int32, 16)
        zeros16 = iota16 * 0
        row_ids = [iota16 + g * 16 for g in range(BPW // 16)]

        row65 = [(iota16 + g * 16) * (D + 1) for g in range(BPW // 16)]

        def transpose_add(l, b):
            src, dst = gb[b], tb[b]
            lD = l * D
            pes4 = [pe_v[pl.ds(lD + q * 16, 16)] for q in range(4)]

            @plsc.parallel_loop(0, BPW, unroll=2)
            def _(r):
                for q in range(4):
                    gp[pl.ds(r * (D + 1) + q * 16, 16)] = (
                        src[r, pl.ds(q * 16, 16)] + pes4[q])

            @plsc.parallel_loop(0, D, unroll=2)
            def _(d):
                d8 = d >> 3
                off = (d & 7) * BPW
                cols = [plsc.load_gather(gp, [row65[g] + d])
                        for g in range(BPW // 16)]
                for g in range(BPW // 16):
                    dst[d8, pl.ds(off + g * 16, 16)] = cols[g]

        for b in range(NBUF):
            start_gather(b, b)

        for b in range(NBUF):
            wait_gather(b, b)
            transpose_add(b, b)
            start_gather(b + NBUF, b)
            start_scatter(b, b)

        @pl.loop(NBUF, SEQ - NBUF, step=NBUF)
        def _(ll):
            for b in range(NBUF):
                l = ll + b
                wait_gather(l, b)
                wait_scatter(l - NBUF, b)
                transpose_add(l, b)
                start_gather(l + NBUF, b)
                start_scatter(l, b)

        for b in range(NBUF):
            l = SEQ - NBUF + b
            wait_gather(l, b)
            wait_scatter(l - NBUF, b)
            transpose_add(l, b)
            start_scatter(l, b)

        for b in range(NBUF):
            wait_scatter(SEQ - NBUF + b, b)

    return k(tpad2, idxt, pe)


_PE = _pe_table(SEQ, D)


def kernel(inputs, table):
    tpad2 = _tc_transpose_pad(table.T).reshape(2 * VOCAB, D)
    idxt = inputs.astype(jnp.int32).T * 2
    pe = jnp.asarray(_PE).reshape(SEQ * D)
    out = _sc_embed(tpad2, idxt, pe)
    out = out.reshape(SEQ, 8, NW, 8, BPW)
    return out.transpose(2, 4, 0, 1, 3).reshape(BATCH, SEQ, D)
